# Initial kernel scaffold; baseline (speedup 1.0000x reference)
import jax
import jax.numpy as jnp
from jax.experimental import pallas as pl


def kernel(call_type_ids, contract_ids, func_selector_ids, depths, exec_properties, call_type_table, contract_table, func_table, depth_table, W_exec, b_exec):
    raise NotImplementedError("write your pallas kernel here")


# speedup vs baseline: 3.1917x; 3.1917x over previous
import functools

import jax
import jax.numpy as jnp
from jax import lax
from jax.experimental import pallas as pl
from jax.experimental.pallas import tpu as pltpu
from jax.experimental.pallas import tpu_sc as plsc

L = 16
TW = 128
CHUNK = 128


def _build_call(N, D, CT_V, DP_V):
    NW = 32
    rows_per_w = N // NW
    n_chunks = rows_per_w // CHUNK
    OUTD = 5 * D

    mesh = plsc.VectorSubcoreMesh(core_axis_name="c", subcore_axis_name="s")

    @functools.partial(
        pl.kernel,
        mesh=mesh,
        compiler_params=pltpu.CompilerParams(needs_layout_passes=False),
        out_type=jax.ShapeDtypeStruct((N, OUTD), jnp.float32),
        scratch_types=[
            pltpu.VMEM((4, CHUNK), jnp.int32),
            pltpu.VMEM((2, CHUNK, TW), jnp.float32),
            pltpu.VMEM((CHUNK, 5 * D), jnp.float32),
            pltpu.VMEM((CHUNK * 4,), jnp.float32),
            pltpu.VMEM((CT_V, D), jnp.float32),
            pltpu.VMEM((DP_V, D), jnp.float32),
            pltpu.VMEM((4 * D,), jnp.float32),
            pltpu.VMEM((D,), jnp.float32),
            pltpu.SemaphoreType.DMA,
        ],
    )
    def call(ids_all, props, ct_tab, co_tab, fn_tab, dp_tab, w_hbm, b_hbm,
             out, idx_v, rows_v, comb_v, p_v, ct_v, dp_v, w_v, b_v, sem):
        wid = lax.axis_index("s") * 2 + lax.axis_index("c")
        pltpu.sync_copy(w_hbm, w_v)
        pltpu.sync_copy(b_hbm, b_v)
        pltpu.sync_copy(ct_tab, ct_v)
        pltpu.sync_copy(dp_tab, dp_v)
        lanes = lax.iota(jnp.int32, L)
        nh = D // L
        wv = [[plsc.load_gather(w_v, [lanes + (f * D + h * L)])
               for h in range(nh)] for f in range(4)]
        bv = [plsc.load_gather(b_v, [lanes + h * L]) for h in range(nh)]
        kf = [jnp.full((L,), k, jnp.int32) for k in range(4)]
        lh = [lanes + h * L for h in range(nh)]
        ch = [[lanes + (k * D + h * L) for h in range(nh)] for k in range(5)]

        def chunk_body(i, carry):
            base = wid * rows_per_w + i * CHUNK
            pltpu.sync_copy(ids_all.at[:, pl.ds(base, CHUNK)], idx_v)
            pltpu.sync_copy(props.at[pl.ds(base * 4, CHUNK * 4)], p_v)
            c_co = pltpu.async_copy(co_tab.at[idx_v.at[1]], rows_v.at[0], sem)
            c_fn = pltpu.async_copy(fn_tab.at[idx_v.at[2]], rows_v.at[1], sem)

            def exec_row(n, c2):
                nn = jnp.full((L,), n, jnp.int32)
                pf = [plsc.load_gather(
                          p_v, [jnp.full((L,), n * 4 + f, jnp.int32)])
                      for f in range(4)]
                for h in range(nh):
                    acc = bv[h]
                    for f in range(4):
                        acc = acc + pf[f] * wv[f][h]
                    acc = jnp.maximum(acc, 0.0)
                    plsc.store_scatter(comb_v, [nn, ch[4][h]], acc)
                ct_id = plsc.load_gather(idx_v, [kf[0], nn])
                dp_id = plsc.load_gather(idx_v, [kf[3], nn])
                for h in range(nh):
                    v = plsc.load_gather(ct_v, [ct_id, lh[h]])
                    plsc.store_scatter(comb_v, [nn, ch[0][h]], v)
                    v = plsc.load_gather(dp_v, [dp_id, lh[h]])
                    plsc.store_scatter(comb_v, [nn, ch[3][h]], v)
                return c2

            lax.fori_loop(0, CHUNK, exec_row, 0)
            c_co.wait()
            c_fn.wait()

            def asm_row(n, c2):
                nn = jnp.full((L,), n, jnp.int32)
                for k in range(2):
                    for h in range(nh):
                        v = plsc.load_gather(rows_v, [kf[k], nn, lh[h]])
                        plsc.store_scatter(comb_v, [nn, ch[1 + k][h]], v)
                return c2

            lax.fori_loop(0, CHUNK, asm_row, 0)
            pltpu.sync_copy(comb_v, out.at[pl.ds(base, CHUNK)])
            return carry

        lax.fori_loop(0, n_chunks, chunk_body, 0)

    return call


def kernel(call_type_ids, contract_ids, func_selector_ids, depths,
           exec_properties, call_type_table, contract_table, func_table,
           depth_table, W_exec, b_exec):
    B, S = call_type_ids.shape
    D = call_type_table.shape[1]
    N = B * S

    ids_all = jnp.stack([
        call_type_ids.reshape(N).astype(jnp.int32),
        contract_ids.reshape(N).astype(jnp.int32),
        func_selector_ids.reshape(N).astype(jnp.int32),
        depths.reshape(N).astype(jnp.int32),
    ])
    co_pad = jnp.pad(contract_table, ((0, 0), (0, TW - D)))
    fn_pad = jnp.pad(func_table, ((0, 0), (0, TW - D)))

    out = _build_call(N, D, call_type_table.shape[0], depth_table.shape[0])(
        ids_all, exec_properties.reshape(N * 4).astype(jnp.float32),
        call_type_table, co_pad, fn_pad, depth_table,
        W_exec.reshape(4 * D), b_exec)
    return out.reshape(B, S, 5 * D)
